# baseline (speedup 1.0000x reference)
import functools
import math

import jax
import jax.numpy as jnp
from jax import lax
from jax.experimental import pallas as pl
from jax.experimental.pallas import tpu as pltpu
from jax.experimental.pallas import tpu_sc as plsc

DIM = 64
SCALE = math.sqrt(DIM)
NC = 2
NS = 16
NW = NC * NS
LANES = 16


@functools.partial(jax.jit, static_argnums=(2,))
def _sc_embed(ids, table, b):
    b_per_w = b // NW
    cb = 512
    n_chunks = b_per_w // cb

    mesh = plsc.VectorSubcoreMesh(core_axis_name="c", subcore_axis_name="s")

    @functools.partial(
        pl.kernel,
        mesh=mesh,
        out_type=jax.ShapeDtypeStruct((b, DIM), jnp.float32),
        scratch_types=[
            pltpu.VMEM((cb,), jnp.int32),
            pltpu.VMEM((cb, DIM), jnp.float32),
            pltpu.SemaphoreType.DMA,
        ],
        compiler_params=pltpu.CompilerParams(use_tc_tiling_on_sc=False),
    )
    def k(ids_hbm, table_hbm, out_hbm, idx_v, rows_v, sem):
        wid = lax.axis_index("s") * NC + lax.axis_index("c")
        base = wid * b_per_w

        def chunk(i, carry):
            off = pl.multiple_of(base + i * cb, 8)
            pltpu.sync_copy(ids_hbm.at[pl.ds(off, cb)], idx_v)
            pltpu.async_copy(table_hbm.at[idx_v], rows_v, sem).wait()

            def scale_row(r, c):
                for j in range(DIM // LANES):
                    s = pl.ds(j * LANES, LANES)
                    rows_v[r, s] = rows_v[r, s] * SCALE
                return c

            lax.fori_loop(0, cb, scale_row, 0)
            pltpu.sync_copy(rows_v, out_hbm.at[pl.ds(off, cb)])
            return carry

        lax.fori_loop(0, n_chunks, chunk, 0)

    return k(ids, table)


def kernel(token_ids, table):
    shp = token_ids.shape
    b = token_ids.size
    ids = token_ids.reshape(-1).astype(jnp.int32)
    out = _sc_embed(ids, table, b)
    return out.reshape(*shp, DIM)


# speedup vs baseline: 1.1336x; 1.1336x over previous
import functools
import math

import jax
import jax.numpy as jnp
from jax import lax
from jax.experimental import pallas as pl
from jax.experimental.pallas import tpu as pltpu
from jax.experimental.pallas import tpu_sc as plsc

DIM = 64
SCALE = math.sqrt(DIM)
NC = 2
NS = 16
NW = NC * NS
LANES = 16
CB = 320
NBUF = 4


@functools.partial(jax.jit, static_argnums=(2,))
def _sc_embed(ids, table, b):
    b_per_w = b // NW
    n_chunks = b_per_w // CB

    mesh = plsc.VectorSubcoreMesh(core_axis_name="c", subcore_axis_name="s")

    @functools.partial(
        pl.kernel,
        mesh=mesh,
        out_type=jax.ShapeDtypeStruct((b, DIM), jnp.float32),
        scratch_types=(
            [pltpu.VMEM((b_per_w,), jnp.int32)]
            + [pltpu.VMEM((CB, DIM), jnp.float32) for _ in range(NBUF)]
            + [pltpu.SemaphoreType.DMA for _ in range(2 * NBUF)]
        ),
        compiler_params=pltpu.CompilerParams(use_tc_tiling_on_sc=False),
    )
    def k(ids_hbm, table_hbm, out_hbm, idx_v, *bufs):
        rows = bufs[:NBUF]
        gsem = bufs[NBUF:2 * NBUF]
        ssem = bufs[2 * NBUF:]
        wid = lax.axis_index("s") * NC + lax.axis_index("c")
        base = wid * b_per_w
        pltpu.sync_copy(ids_hbm.at[pl.ds(base, b_per_w)], idx_v)

        def gather(chunk, buf):
            pltpu.async_copy(
                table_hbm.at[idx_v.at[pl.ds(chunk * CB, CB)]],
                rows[buf], gsem[buf])

        def wait_gather(buf):
            pltpu.make_async_copy(
                table_hbm.at[pl.ds(0, CB)], rows[buf], gsem[buf]).wait()

        def wait_store(buf):
            pltpu.make_async_copy(
                rows[buf], out_hbm.at[pl.ds(0, CB)], ssem[buf]).wait()

        for g in range(NBUF - 1):
            gather(g, g)

        def outer(gi, carry):
            for bb in range(NBUF):
                g = gi * NBUF + bb
                wait_gather(bb)

                @plsc.parallel_loop(0, CB, step=1, unroll=8)
                def scale_row(r):
                    for j in range(DIM // LANES):
                        s = pl.ds(j * LANES, LANES)
                        rows[bb][r, s] = rows[bb][r, s] * SCALE

                off = pl.multiple_of(base + g * CB, 8)
                pltpu.async_copy(rows[bb], out_hbm.at[pl.ds(off, CB)],
                                 ssem[bb])
                bp = (bb - 1) % NBUF

                @pl.when(g > 0)
                def _():
                    wait_store(bp)

                @pl.when(g + NBUF - 1 < n_chunks)
                def _():
                    gather(g + NBUF - 1, bp)
            return carry

        lax.fori_loop(0, n_chunks // NBUF, outer, 0)
        wait_store((n_chunks - 1) % NBUF)

    return k(ids, table)


def kernel(token_ids, table):
    shp = token_ids.shape
    b = token_ids.size
    ids = token_ids.reshape(-1).astype(jnp.int32)
    out = _sc_embed(ids, table, b)
    return out.reshape(*shp, DIM)
